# baseline (speedup 1.0000x reference)
import jax
import jax.numpy as jnp
from jax.experimental import pallas as pl
from jax.experimental.pallas import tpu as pltpu

E = 8
D = 1024
F = 768
T = 2048
FC = 2
FH = F // FC


def _router_body(x_ref, wg_ref, wfull_ref, xb_ref):
    x = x_ref[...].astype(jnp.bfloat16)
    xb_ref[...] = x
    gw = wg_ref[...].astype(jnp.bfloat16)
    logits = jax.lax.dot_general(
        x, gw, (((1,), (1,)), ((), ())),
        preferred_element_type=jnp.float32,
    )
    m = jnp.max(logits, axis=1, keepdims=True)
    p = jnp.exp(logits - m)
    p = p / jnp.sum(p, axis=1, keepdims=True)
    lane = jax.lax.broadcasted_iota(jnp.int32, (T, E), 1)
    m1 = jnp.max(p, axis=1, keepdims=True)
    a1 = jnp.min(jnp.where(p == m1, lane, E), axis=1, keepdims=True)
    pm = jnp.where(lane == a1, -1.0, p)
    m2 = jnp.max(pm, axis=1, keepdims=True)
    a2 = jnp.min(jnp.where(pm == m2, lane, E), axis=1, keepdims=True)
    one1 = lane == a1
    one2 = lane == a2
    s = m1 + m2
    wfull_ref[...] = (jnp.where(one1, m1, 0.0) + jnp.where(one2, m2, 0.0)) / s


TCH = 256


def _ffn_body(xb_ref, wg_ref, wu_ref, wd_ref, wf_ref, out_ref):
    e = pl.program_id(0)
    wg = wg_ref[0].astype(jnp.bfloat16)
    wu = wu_ref[0].astype(jnp.bfloat16)
    wd = wd_ref[0].astype(jnp.bfloat16)
    lane = jax.lax.broadcasted_iota(jnp.int32, (TCH, E), 1)
    for tc in range(T // TCH):
        sl = pl.ds(tc * TCH, TCH)
        xb = xb_ref[sl, :]
        g = jnp.dot(xb, wg, preferred_element_type=jnp.float32)
        u = jnp.dot(xb, wu, preferred_element_type=jnp.float32)
        h = (g * jax.nn.sigmoid(g)) * u
        y = jnp.dot(h.astype(jnp.bfloat16), wd,
                    preferred_element_type=jnp.float32)
        we = jnp.sum(jnp.where(lane == e, wf_ref[sl, :], 0.0), axis=1)
        contrib = y * we[:, None]

        @pl.when(e == 0)
        def _():
            out_ref[sl, :] = contrib

        @pl.when(e > 0)
        def _():
            out_ref[sl, :] = out_ref[sl, :] + contrib


@jax.jit
def kernel(hidden_states, W_gate, W_g, W_u, W_d):
    orig_shape = hidden_states.shape
    x = hidden_states.reshape(T, D)
    wfull, xb = pl.pallas_call(
        _router_body,
        out_shape=(
            jax.ShapeDtypeStruct((T, E), jnp.float32),
            jax.ShapeDtypeStruct((T, D), jnp.bfloat16),
        ),
    )(x, W_gate)

    out = pl.pallas_call(
        _ffn_body,
        grid=(E,),
        in_specs=[
            pl.BlockSpec((T, D), lambda e: (0, 0)),
            pl.BlockSpec((1, D, F), lambda e: (e, 0, 0)),
            pl.BlockSpec((1, D, F), lambda e: (e, 0, 0)),
            pl.BlockSpec((1, F, D), lambda e: (e, 0, 0)),
            pl.BlockSpec((T, E), lambda e: (0, 0)),
        ],
        out_specs=pl.BlockSpec((T, D), lambda e: (0, 0)),
        out_shape=jax.ShapeDtypeStruct((T, D), jnp.float32),
    )(xb, W_g, W_u, W_d, wfull)
    return out.reshape(orig_shape)


# speedup vs baseline: 1.0703x; 1.0703x over previous
import jax
import jax.numpy as jnp
from jax.experimental import pallas as pl
E = 8
D = 1024
F = 768
T = 2048


def _router_body(x_ref, wg_ref, wfull_ref, xb_ref):
    x = x_ref[...].astype(jnp.bfloat16)
    xb_ref[...] = x
    gw = wg_ref[...].astype(jnp.bfloat16)
    logits = jax.lax.dot_general(
        x, gw, (((1,), (1,)), ((), ())),
        preferred_element_type=jnp.float32,
    )
    m = jnp.max(logits, axis=1, keepdims=True)
    p = jnp.exp(logits - m)
    p = p / jnp.sum(p, axis=1, keepdims=True)
    lane = jax.lax.broadcasted_iota(jnp.int32, (T, E), 1)
    m1 = jnp.max(p, axis=1, keepdims=True)
    a1 = jnp.min(jnp.where(p == m1, lane, E), axis=1, keepdims=True)
    pm = jnp.where(lane == a1, -1.0, p)
    m2 = jnp.max(pm, axis=1, keepdims=True)
    a2 = jnp.min(jnp.where(pm == m2, lane, E), axis=1, keepdims=True)
    one1 = lane == a1
    one2 = lane == a2
    s = m1 + m2
    wfull_ref[...] = (jnp.where(one1, m1, 0.0) + jnp.where(one2, m2, 0.0)) / s


TCH = 512


def _ffn_body(xb_ref, wg_ref, wu_ref, wd_ref, wf_ref, out_ref):
    e = pl.program_id(0)
    wg = wg_ref[0].astype(jnp.bfloat16)
    wu = wu_ref[0].astype(jnp.bfloat16)
    wd = wd_ref[0].astype(jnp.bfloat16)
    lane = jax.lax.broadcasted_iota(jnp.int32, (TCH, E), 1)
    for tc in range(T // TCH):
        sl = pl.ds(tc * TCH, TCH)
        xb = xb_ref[sl, :]
        g = jnp.dot(xb, wg, preferred_element_type=jnp.float32)
        u = jnp.dot(xb, wu, preferred_element_type=jnp.float32)
        h = (g * jax.nn.sigmoid(g)) * u
        y = jnp.dot(h.astype(jnp.bfloat16), wd,
                    preferred_element_type=jnp.float32)
        we = jnp.sum(jnp.where(lane == e, wf_ref[sl, :], 0.0), axis=1)
        contrib = y * we[:, None]

        @pl.when(e == 0)
        def _():
            out_ref[sl, :] = contrib

        @pl.when(e > 0)
        def _():
            out_ref[sl, :] = out_ref[sl, :] + contrib


@jax.jit
def kernel(hidden_states, W_gate, W_g, W_u, W_d):
    orig_shape = hidden_states.shape
    x = hidden_states.reshape(T, D)
    wfull, xb = pl.pallas_call(
        _router_body,
        out_shape=(
            jax.ShapeDtypeStruct((T, E), jnp.float32),
            jax.ShapeDtypeStruct((T, D), jnp.bfloat16),
        ),
    )(x, W_gate)

    out = pl.pallas_call(
        _ffn_body,
        grid=(E,),
        in_specs=[
            pl.BlockSpec((T, D), lambda e: (0, 0)),
            pl.BlockSpec((1, D, F), lambda e: (e, 0, 0)),
            pl.BlockSpec((1, D, F), lambda e: (e, 0, 0)),
            pl.BlockSpec((1, F, D), lambda e: (e, 0, 0)),
            pl.BlockSpec((T, E), lambda e: (0, 0)),
        ],
        out_specs=pl.BlockSpec((T, D), lambda e: (0, 0)),
        out_shape=jax.ShapeDtypeStruct((T, D), jnp.float32),
    )(xb, W_g, W_u, W_d, wfull)
    return out.reshape(orig_shape)
